# baseline (speedup 1.0000x reference)
import functools

import jax
import jax.numpy as jnp
import numpy as np
from jax import lax
from jax.experimental import pallas as pl
from jax.experimental.pallas import tpu as pltpu
from jax.experimental.pallas import tpu_sc as plsc

N = 320000
D = 128
NSEG = 10000
GAMMA = 0.01
INVG = float(np.float32(1.0) / np.float32(GAMMA))

NC = 2
NS = 16
NW = NC * NS
RPW = N // NW

C = 256
SHIFT = 100.0
SCAP = 64
WIN = 2048
NEG = -1e30
NVEC = 8


def _sc_body(v_hbm, seg_hbm, m_hbm, s_hbm, rowb, idb, rowb2, idb2, idwin,
             id8, mstage, sstage, bounds, semr0, semi0, semr1, semi1):
    wid = lax.axis_index("c") * NS + lax.axis_index("s")

    negv = jnp.full((16,), NEG, jnp.float32)

    def scan_window(p, b):
        p = pl.multiple_of(p, 8)
        pltpu.sync_copy(seg_hbm.at[pl.ds(p, WIN)], idwin)
        big = jnp.int32(10**9)

        def group(off, found):
            va = idwin[pl.ds(off, 16)]
            vb = idwin[pl.ds(off + 1, 16)]
            neq = va != vb
            io = lax.iota(jnp.int32, 16)
            cand = jnp.min(jnp.where(neq, io, big))
            pos = p + off + cand + 1
            ok = (cand < big) & (pos >= b)
            return jnp.where(ok, jnp.minimum(found, pos), found)

        def grp_k(k, found):
            return group(k * 16, found)

        found = lax.fori_loop(0, (WIN - 17) // 16 + 1, grp_k, jnp.int32(N))
        return group(jnp.int32(WIN - 17), found)

    def find_change(b):
        p0 = jnp.minimum(b - 8, N - WIN)
        f0 = scan_window(p0, b)

        def cond(st):
            return (st[1] == N) & (st[0] < N - WIN)

        def step(st):
            pn = jnp.minimum(st[0] + (WIN - 8), N - WIN)
            return (pn, scan_window(pn, b))

        return lax.while_loop(cond, step, (p0, f0))[1]

    b_lo = jnp.where(wid == 0, 8, wid * RPW)
    b_hi = jnp.where(wid == NW - 1, 8, (wid + 1) * RPW)
    a_start = jnp.where(wid == 0, 0, find_change(b_lo))
    a_end = jnp.where(wid == NW - 1, N, find_change(b_hi))

    def seg_at(r):
        r = jnp.minimum(r, N - 1)
        r8 = pl.multiple_of(jnp.minimum((r // 8) * 8, N - 16), 8)
        pltpu.sync_copy(seg_hbm.at[pl.ds(r8, 16)], id8.at[pl.ds(0, 16)])
        return id8[pl.ds(r - r8, 16)][0]

    own_lo = jnp.where(wid == 0, 0,
                       jnp.where(a_start >= N, NSEG, seg_at(a_start)))
    own_hi = jnp.where((wid == NW - 1) | (a_end >= N), NSEG, seg_at(a_end))

    def flush_full(base):
        o = pl.multiple_of(base * D, 8)
        pltpu.sync_copy(mstage, m_hbm.at[pl.ds(o, SCAP * D)])
        pltpu.sync_copy(sstage, s_hbm.at[pl.ds(o, SCAP * D)])

    def maybe_flush(cnt, base):
        def f(st):
            flush_full(st[1])
            return (jnp.int32(0), st[1] + SCAP)

        return lax.cond(cnt == SCAP, f, lambda st: st, (cnt, base))

    def write_identity(cnt):
        for j in range(NVEC):
            mstage[pl.ds(cnt * D + j * 16, 16)] = negv
            sstage[pl.ds(cnt * D + j * 16, 16)] = jnp.zeros((16,), jnp.float32)

    def fill_to(tid, cnt, base):
        def gcond(st):
            return st[0] + st[1] < tid

        def gbody(st):
            write_identity(st[0])
            return maybe_flush(st[0] + 1, st[1])

        return lax.while_loop(gcond, gbody, (cnt, base))

    def stage_put(sid, ms, ss, cnt, base):
        cnt, base = fill_to(sid, cnt, base)
        for j in range(NVEC):
            mstage[pl.ds(cnt * D + j * 16, 16)] = ms[j]
            sstage[pl.ds(cnt * D + j * 16, 16)] = ss[j]
        return maybe_flush(cnt + 1, base)

    A0 = (a_start // 8) * 8
    nchunks = (a_end - A0 + C - 1) // C
    zerov = jnp.zeros((16,), jnp.float32)

    def chunk_base(k):
        return pl.multiple_of(jnp.minimum(A0 + k * C, N - C), 8)

    def issue(k, rb, ib, semr, semi):
        r0c = chunk_base(k)
        pltpu.async_copy(v_hbm.at[pl.ds(r0c, C)], rb, semr)
        pltpu.async_copy(seg_hbm.at[pl.ds(r0c, C)], ib.at[pl.ds(0, C)], semi)

    def wait_dma(k, rb, ib, semr, semi):
        r0c = chunk_base(k)
        pltpu.make_async_copy(v_hbm.at[pl.ds(r0c, C)], rb, semr).wait()
        pltpu.make_async_copy(seg_hbm.at[pl.ds(r0c, C)], ib.at[pl.ds(0, C)],
                              semi).wait()

    def process_chunk(k, rb, ib, carry):
        r0 = A0 + k * C
        r0c = chunk_base(k)
        lo = jnp.maximum(a_start - r0c, r0 - r0c)
        hi = jnp.minimum(a_end - r0c, jnp.int32(C))

        def grp(g, nb):
            va = ib[pl.ds(g * 16, 16)]
            vb = ib[pl.ds(g * 16 + 1, 16)]
            msk = va != vb
            pos = lax.iota(jnp.int32, 16) + (g * 16 + 1)
            plsc.store_compressed(bounds.at[pl.ds(nb, 16)], pos, mask=msk)
            return nb + plsc.all_reduce_population_count(msk)[0]

        nb = lax.fori_loop(0, C // 16, grp, jnp.int32(0))

        def piece_body(q, cy):
            prev = cy[-1]
            b_q = jnp.where(q < nb, bounds[pl.ds(q, 16)][0], hi)
            s0 = jnp.maximum(prev, lo)
            e0 = jnp.minimum(b_q, hi)

            def do_piece(op):
                ms = op[0:NVEC]
                ss = op[NVEC:2 * NVEC]
                cur, cnt, base = op[2 * NVEC], op[2 * NVEC + 1], op[2 * NVEC + 2]
                pid = ib[pl.ds(s0, 16)][0]

                def fast1(i, st):
                    acc = st[0:NVEC]
                    mm = st[NVEC:2 * NVEC]
                    row = [rb[i, pl.ds(j * 16, 16)] for j in range(NVEC)]
                    return tuple(
                        acc[j] + jnp.exp(row[j] * INVG - SHIFT)
                        for j in range(NVEC)) + tuple(
                        jnp.maximum(mm[j], row[j]) for j in range(NVEC))

                def fast4(ii, st):
                    i = s0 + ii * 4
                    for u in range(4):
                        st = fast1(i + u, st)
                    return st

                nrows = e0 - s0
                st = lax.fori_loop(0, nrows // 4, fast4,
                                   (zerov,) * NVEC + (negv,) * NVEC)
                st = lax.fori_loop(s0 + (nrows // 4) * 4, e0, fast1, st)
                acc = st[0:NVEC]
                ml = [st[NVEC + j] * INVG for j in range(NVEC)]
                mtree = jnp.minimum(
                    jnp.minimum(jnp.minimum(ml[0], ml[1]),
                                jnp.minimum(ml[2], ml[3])),
                    jnp.minimum(jnp.minimum(ml[4], ml[5]),
                                jnp.minimum(ml[6], ml[7])))
                min_m = jnp.min(mtree)
                xtree = jnp.maximum(
                    jnp.maximum(jnp.maximum(ml[0], ml[1]),
                                jnp.maximum(ml[2], ml[3])),
                    jnp.maximum(jnp.maximum(ml[4], ml[5]),
                                jnp.maximum(ml[6], ml[7])))
                max_m = jnp.max(xtree)

                def rescale(_):
                    return tuple(acc[j] * jnp.exp(SHIFT - ml[j])
                                 for j in range(NVEC))

                def redo(_):
                    def p2(i, a2):
                        return tuple(
                            a2[j] + jnp.exp(rb[i, pl.ds(j * 16, 16)] * INVG
                                            - ml[j])
                            for j in range(NVEC))

                    return lax.fori_loop(s0, e0, p2, (zerov,) * NVEC)

                sl = lax.cond((min_m >= 13.0) & (max_m <= SHIFT),
                              rescale, redo, 0)

                def merge(st):
                    cnt2, base2 = st
                    mn = [jnp.maximum(ms[j], ml[j]) for j in range(NVEC)]
                    sn = [ss[j] * jnp.exp(ms[j] - mn[j])
                          + sl[j] * jnp.exp(ml[j] - mn[j])
                          for j in range(NVEC)]
                    return tuple(mn) + tuple(sn) + (cnt2, base2)

                def flush_rep(st):
                    cnt2, base2 = lax.cond(
                        cur >= 0,
                        lambda st2: stage_put(cur, ms, ss, st2[0], st2[1]),
                        lambda st2: st2,
                        st,
                    )
                    return tuple(ml) + tuple(sl) + (cnt2, base2)

                out = lax.cond(pid == cur, merge, flush_rep, (cnt, base))
                return out[:2 * NVEC] + (pid,) + out[2 * NVEC:]

            core = lax.cond(e0 > s0, do_piece, lambda op: op, cy[:-1])
            return core + (b_q,)

        cy = lax.fori_loop(0, nb + 1, piece_body, carry + (jnp.int32(0),))
        return cy[:-1]

    carry0 = (negv,) * NVEC + (zerov,) * NVEC + (
        jnp.int32(-1), jnp.int32(0), own_lo)

    npairs = (nchunks + 1) // 2
    issue(jnp.int32(0), rowb, idb, semr0, semi0)

    def pair_body(g, carry):
        k0 = 2 * g
        issue(k0 + 1, rowb2, idb2, semr1, semi1)
        wait_dma(k0, rowb, idb, semr0, semi0)
        carry = process_chunk(k0, rowb, idb, carry)
        issue(k0 + 2, rowb, idb, semr0, semi0)
        wait_dma(k0 + 1, rowb2, idb2, semr1, semi1)
        carry = process_chunk(k0 + 1, rowb2, idb2, carry)
        return carry

    carry = lax.fori_loop(0, npairs, pair_body, carry0)
    wait_dma(2 * npairs, rowb, idb, semr0, semi0)

    ms = carry[0:NVEC]
    ss = carry[NVEC:2 * NVEC]
    cur, cnt, base = carry[2 * NVEC], carry[2 * NVEC + 1], carry[2 * NVEC + 2]

    cnt, base = lax.cond(
        cur >= 0,
        lambda st: stage_put(cur, ms, ss, st[0], st[1]),
        lambda st: st,
        (cnt, base),
    )
    cnt, base = fill_to(own_hi, cnt, base)

    def blk_cond(off):
        return off + 8 <= cnt

    def blk_body(off):
        so = pl.multiple_of(off * D, 8)
        do = pl.multiple_of((base + off) * D, 8)
        pltpu.sync_copy(mstage.at[pl.ds(so, 8 * D)], m_hbm.at[pl.ds(do, 8 * D)])
        pltpu.sync_copy(sstage.at[pl.ds(so, 8 * D)], s_hbm.at[pl.ds(do, 8 * D)])
        return off + 8

    off = lax.while_loop(blk_cond, blk_body, jnp.int32(0))

    def one_cond(off):
        return off < cnt

    def one_body(off):
        so = pl.multiple_of(off * D, 8)
        do = pl.multiple_of((base + off) * D, 8)
        pltpu.sync_copy(mstage.at[pl.ds(so, D)], m_hbm.at[pl.ds(do, D)])
        pltpu.sync_copy(sstage.at[pl.ds(so, D)], s_hbm.at[pl.ds(do, D)])
        return off + 1

    lax.while_loop(one_cond, one_body, off)


_sc_call = functools.partial(
    pl.kernel,
    out_type=[
        jax.ShapeDtypeStruct((NSEG * D,), jnp.float32),
        jax.ShapeDtypeStruct((NSEG * D,), jnp.float32),
    ],
    mesh=plsc.VectorSubcoreMesh(
        core_axis_name="c", subcore_axis_name="s", num_cores=NC,
        num_subcores=NS),
    compiler_params=pltpu.CompilerParams(needs_layout_passes=False),
    scratch_types=[
        pltpu.VMEM((C, D), jnp.float32),
        pltpu.VMEM((C + 16,), jnp.int32),
        pltpu.VMEM((C, D), jnp.float32),
        pltpu.VMEM((C + 16,), jnp.int32),
        pltpu.VMEM((WIN,), jnp.int32),
        pltpu.VMEM((32,), jnp.int32),
        pltpu.VMEM((SCAP * D,), jnp.float32),
        pltpu.VMEM((SCAP * D,), jnp.float32),
        pltpu.VMEM((C + 32,), jnp.int32),
        pltpu.SemaphoreType.DMA,
        pltpu.SemaphoreType.DMA,
        pltpu.SemaphoreType.DMA,
        pltpu.SemaphoreType.DMA,
    ],
)(_sc_body)


def _fin_body(m_ref, s_ref, o_ref):
    m = m_ref[...]
    s = s_ref[...]
    m0 = jnp.where(m < -1e29, 0.0, m)
    o_ref[...] = jnp.float32(GAMMA) * (m0 + jnp.log(s + 1e-12))


_finalize = pl.pallas_call(
    _fin_body,
    grid=(10,),
    in_specs=[
        pl.BlockSpec((NSEG // 10, D), lambda i: (i, 0)),
        pl.BlockSpec((NSEG // 10, D), lambda i: (i, 0)),
    ],
    out_specs=pl.BlockSpec((NSEG // 10, D), lambda i: (i, 0)),
    out_shape=jax.ShapeDtypeStruct((NSEG, D), jnp.float32),
)


def kernel(v, segment_ids):
    seg = segment_ids.astype(jnp.int32)
    m, s = _sc_call(v, seg)
    return _finalize(m.reshape(NSEG, D), s.reshape(NSEG, D))


# speedup vs baseline: 1.0135x; 1.0135x over previous
import functools

import jax
import jax.numpy as jnp
import numpy as np
from jax import lax
from jax.experimental import pallas as pl
from jax.experimental.pallas import tpu as pltpu
from jax.experimental.pallas import tpu_sc as plsc

N = 320000
D = 128
NSEG = 10000
GAMMA = 0.01
INVG = float(np.float32(1.0) / np.float32(GAMMA))

NC = 2
NS = 16
NW = NC * NS
RPW = N // NW

C = 256
SHIFT = 100.0
SCAP = 64
WIN = 2048
NEG = -1e30
NVEC = 8


def _sc_body(v_hbm, seg_hbm, m_hbm, s_hbm, rowb, idb, rowb2, idb2, idwin,
             id8, mstage, sstage, bounds, semr0, semi0, semr1, semi1):
    wid = lax.axis_index("c") * NS + lax.axis_index("s")

    negv = jnp.full((16,), NEG, jnp.float32)

    def scan_window(p, b):
        p = pl.multiple_of(p, 8)
        pltpu.sync_copy(seg_hbm.at[pl.ds(p, WIN)], idwin)
        big = jnp.int32(10**9)

        def group(off, found):
            va = idwin[pl.ds(off, 16)]
            vb = idwin[pl.ds(off + 1, 16)]
            neq = va != vb
            io = lax.iota(jnp.int32, 16)
            cand = jnp.min(jnp.where(neq, io, big))
            pos = p + off + cand + 1
            ok = (cand < big) & (pos >= b)
            return jnp.where(ok, jnp.minimum(found, pos), found)

        def grp_k(k, found):
            return group(k * 16, found)

        found = lax.fori_loop(0, (WIN - 17) // 16 + 1, grp_k, jnp.int32(N))
        return group(jnp.int32(WIN - 17), found)

    def find_change(b):
        p0 = jnp.minimum(b - 8, N - WIN)
        f0 = scan_window(p0, b)

        def cond(st):
            return (st[1] == N) & (st[0] < N - WIN)

        def step(st):
            pn = jnp.minimum(st[0] + (WIN - 8), N - WIN)
            return (pn, scan_window(pn, b))

        return lax.while_loop(cond, step, (p0, f0))[1]

    b_lo = jnp.where(wid == 0, 8, wid * RPW)
    b_hi = jnp.where(wid == NW - 1, 8, (wid + 1) * RPW)
    a_start = jnp.where(wid == 0, 0, find_change(b_lo))
    a_end = jnp.where(wid == NW - 1, N, find_change(b_hi))

    def seg_at(r):
        r = jnp.minimum(r, N - 1)
        r8 = pl.multiple_of(jnp.minimum((r // 8) * 8, N - 16), 8)
        pltpu.sync_copy(seg_hbm.at[pl.ds(r8, 16)], id8.at[pl.ds(0, 16)])
        return id8[pl.ds(r - r8, 16)][0]

    own_lo = jnp.where(wid == 0, 0,
                       jnp.where(a_start >= N, NSEG, seg_at(a_start)))
    own_hi = jnp.where((wid == NW - 1) | (a_end >= N), NSEG, seg_at(a_end))

    def flush_full(base):
        o = pl.multiple_of(base * D, 8)
        pltpu.sync_copy(mstage, m_hbm.at[pl.ds(o, SCAP * D)])
        pltpu.sync_copy(sstage, s_hbm.at[pl.ds(o, SCAP * D)])

    def maybe_flush(cnt, base):
        def f(st):
            flush_full(st[1])
            return (jnp.int32(0), st[1] + SCAP)

        return lax.cond(cnt == SCAP, f, lambda st: st, (cnt, base))

    def write_identity(cnt):
        for j in range(NVEC):
            mstage[pl.ds(cnt * D + j * 16, 16)] = negv
            sstage[pl.ds(cnt * D + j * 16, 16)] = jnp.zeros((16,), jnp.float32)

    def fill_to(tid, cnt, base):
        def gcond(st):
            return st[0] + st[1] < tid

        def gbody(st):
            write_identity(st[0])
            return maybe_flush(st[0] + 1, st[1])

        return lax.while_loop(gcond, gbody, (cnt, base))

    def stage_put(sid, ms, ss, cnt, base):
        cnt, base = fill_to(sid, cnt, base)
        for j in range(NVEC):
            mstage[pl.ds(cnt * D + j * 16, 16)] = ms[j]
            sstage[pl.ds(cnt * D + j * 16, 16)] = ss[j]
        return maybe_flush(cnt + 1, base)

    A0 = (a_start // 8) * 8
    nchunks = (a_end - A0 + C - 1) // C
    zerov = jnp.zeros((16,), jnp.float32)

    def chunk_base(k):
        return pl.multiple_of(jnp.minimum(A0 + k * C, N - C), 8)

    def issue(k, rb, ib, semr, semi):
        r0c = chunk_base(k)
        pltpu.async_copy(v_hbm.at[pl.ds(r0c, C)], rb, semr)
        pltpu.async_copy(seg_hbm.at[pl.ds(r0c, C)], ib.at[pl.ds(0, C)], semi)

    def wait_dma(k, rb, ib, semr, semi):
        r0c = chunk_base(k)
        pltpu.make_async_copy(v_hbm.at[pl.ds(r0c, C)], rb, semr).wait()
        pltpu.make_async_copy(seg_hbm.at[pl.ds(r0c, C)], ib.at[pl.ds(0, C)],
                              semi).wait()

    def process_chunk(k, rb, ib, carry):
        r0 = A0 + k * C
        r0c = chunk_base(k)
        lo = jnp.maximum(a_start - r0c, r0 - r0c)
        hi = jnp.minimum(a_end - r0c, jnp.int32(C))

        def grp(g, nb):
            va = ib[pl.ds(g * 16, 16)]
            vb = ib[pl.ds(g * 16 + 1, 16)]
            msk = va != vb
            pos = lax.iota(jnp.int32, 16) + (g * 16 + 1)
            plsc.store_compressed(bounds.at[pl.ds(nb, 16)], pos, mask=msk)
            return nb + plsc.all_reduce_population_count(msk)[0]

        nb = lax.fori_loop(0, C // 16, grp, jnp.int32(0))

        def piece_body(q, cy):
            prev = cy[-1]
            b_q = jnp.where(q < nb, bounds[pl.ds(q, 16)][0], hi)
            s0 = jnp.maximum(prev, lo)
            e0 = jnp.minimum(b_q, hi)

            def do_piece(op):
                ms = op[0:NVEC]
                ss = op[NVEC:2 * NVEC]
                cur, cnt, base = op[2 * NVEC], op[2 * NVEC + 1], op[2 * NVEC + 2]
                pid = ib[pl.ds(s0, 16)][0]

                def fast1(i, st):
                    acc = st[0:NVEC]
                    mm = st[NVEC:2 * NVEC]
                    row = [rb[i, pl.ds(j * 16, 16)] for j in range(NVEC)]
                    return tuple(
                        acc[j] + jnp.exp(row[j] * INVG - SHIFT)
                        for j in range(NVEC)) + tuple(
                        jnp.maximum(mm[j], row[j]) for j in range(NVEC))

                def fast2(ii, st):
                    i = s0 + ii * 2
                    return fast1(i + 1, fast1(i, st))

                nrows = e0 - s0
                st = lax.fori_loop(0, nrows // 2, fast2,
                                   (zerov,) * NVEC + (negv,) * NVEC)
                st = lax.fori_loop(s0 + (nrows // 2) * 2, e0, fast1, st)
                acc = st[0:NVEC]
                ml = [st[NVEC + j] * INVG for j in range(NVEC)]
                mtree = jnp.minimum(
                    jnp.minimum(jnp.minimum(ml[0], ml[1]),
                                jnp.minimum(ml[2], ml[3])),
                    jnp.minimum(jnp.minimum(ml[4], ml[5]),
                                jnp.minimum(ml[6], ml[7])))
                min_m = jnp.min(mtree)
                xtree = jnp.maximum(
                    jnp.maximum(jnp.maximum(ml[0], ml[1]),
                                jnp.maximum(ml[2], ml[3])),
                    jnp.maximum(jnp.maximum(ml[4], ml[5]),
                                jnp.maximum(ml[6], ml[7])))
                max_m = jnp.max(xtree)

                def rescale(_):
                    return tuple(acc[j] * jnp.exp(SHIFT - ml[j])
                                 for j in range(NVEC))

                def redo(_):
                    def p2(i, a2):
                        return tuple(
                            a2[j] + jnp.exp(rb[i, pl.ds(j * 16, 16)] * INVG
                                            - ml[j])
                            for j in range(NVEC))

                    return lax.fori_loop(s0, e0, p2, (zerov,) * NVEC)

                sl = lax.cond((min_m >= 13.0) & (max_m <= SHIFT),
                              rescale, redo, 0)

                def merge(st):
                    cnt2, base2 = st
                    mn = [jnp.maximum(ms[j], ml[j]) for j in range(NVEC)]
                    sn = [ss[j] * jnp.exp(ms[j] - mn[j])
                          + sl[j] * jnp.exp(ml[j] - mn[j])
                          for j in range(NVEC)]
                    return tuple(mn) + tuple(sn) + (cnt2, base2)

                def flush_rep(st):
                    cnt2, base2 = lax.cond(
                        cur >= 0,
                        lambda st2: stage_put(cur, ms, ss, st2[0], st2[1]),
                        lambda st2: st2,
                        st,
                    )
                    return tuple(ml) + tuple(sl) + (cnt2, base2)

                out = lax.cond(pid == cur, merge, flush_rep, (cnt, base))
                return out[:2 * NVEC] + (pid,) + out[2 * NVEC:]

            core = lax.cond(e0 > s0, do_piece, lambda op: op, cy[:-1])
            return core + (b_q,)

        cy = lax.fori_loop(0, nb + 1, piece_body, carry + (jnp.int32(0),))
        return cy[:-1]

    carry0 = (negv,) * NVEC + (zerov,) * NVEC + (
        jnp.int32(-1), jnp.int32(0), own_lo)

    npairs = (nchunks + 1) // 2
    issue(jnp.int32(0), rowb, idb, semr0, semi0)

    def pair_body(g, carry):
        k0 = 2 * g
        issue(k0 + 1, rowb2, idb2, semr1, semi1)
        wait_dma(k0, rowb, idb, semr0, semi0)
        carry = process_chunk(k0, rowb, idb, carry)
        issue(k0 + 2, rowb, idb, semr0, semi0)
        wait_dma(k0 + 1, rowb2, idb2, semr1, semi1)
        carry = process_chunk(k0 + 1, rowb2, idb2, carry)
        return carry

    carry = lax.fori_loop(0, npairs, pair_body, carry0)
    wait_dma(2 * npairs, rowb, idb, semr0, semi0)

    ms = carry[0:NVEC]
    ss = carry[NVEC:2 * NVEC]
    cur, cnt, base = carry[2 * NVEC], carry[2 * NVEC + 1], carry[2 * NVEC + 2]

    cnt, base = lax.cond(
        cur >= 0,
        lambda st: stage_put(cur, ms, ss, st[0], st[1]),
        lambda st: st,
        (cnt, base),
    )
    cnt, base = fill_to(own_hi, cnt, base)

    def blk_cond(off):
        return off + 8 <= cnt

    def blk_body(off):
        so = pl.multiple_of(off * D, 8)
        do = pl.multiple_of((base + off) * D, 8)
        pltpu.sync_copy(mstage.at[pl.ds(so, 8 * D)], m_hbm.at[pl.ds(do, 8 * D)])
        pltpu.sync_copy(sstage.at[pl.ds(so, 8 * D)], s_hbm.at[pl.ds(do, 8 * D)])
        return off + 8

    off = lax.while_loop(blk_cond, blk_body, jnp.int32(0))

    def one_cond(off):
        return off < cnt

    def one_body(off):
        so = pl.multiple_of(off * D, 8)
        do = pl.multiple_of((base + off) * D, 8)
        pltpu.sync_copy(mstage.at[pl.ds(so, D)], m_hbm.at[pl.ds(do, D)])
        pltpu.sync_copy(sstage.at[pl.ds(so, D)], s_hbm.at[pl.ds(do, D)])
        return off + 1

    lax.while_loop(one_cond, one_body, off)


_sc_call = functools.partial(
    pl.kernel,
    out_type=[
        jax.ShapeDtypeStruct((NSEG * D,), jnp.float32),
        jax.ShapeDtypeStruct((NSEG * D,), jnp.float32),
    ],
    mesh=plsc.VectorSubcoreMesh(
        core_axis_name="c", subcore_axis_name="s", num_cores=NC,
        num_subcores=NS),
    compiler_params=pltpu.CompilerParams(needs_layout_passes=False),
    scratch_types=[
        pltpu.VMEM((C, D), jnp.float32),
        pltpu.VMEM((C + 16,), jnp.int32),
        pltpu.VMEM((C, D), jnp.float32),
        pltpu.VMEM((C + 16,), jnp.int32),
        pltpu.VMEM((WIN,), jnp.int32),
        pltpu.VMEM((32,), jnp.int32),
        pltpu.VMEM((SCAP * D,), jnp.float32),
        pltpu.VMEM((SCAP * D,), jnp.float32),
        pltpu.VMEM((C + 32,), jnp.int32),
        pltpu.SemaphoreType.DMA,
        pltpu.SemaphoreType.DMA,
        pltpu.SemaphoreType.DMA,
        pltpu.SemaphoreType.DMA,
    ],
)(_sc_body)


def _fin_body(m_ref, s_ref, o_ref):
    m = m_ref[...]
    s = s_ref[...]
    m0 = jnp.where(m < -1e29, 0.0, m)
    o_ref[...] = jnp.float32(GAMMA) * (m0 + jnp.log(s + 1e-12))


_finalize = pl.pallas_call(
    _fin_body,
    grid=(10,),
    in_specs=[
        pl.BlockSpec((NSEG // 10, D), lambda i: (i, 0)),
        pl.BlockSpec((NSEG // 10, D), lambda i: (i, 0)),
    ],
    out_specs=pl.BlockSpec((NSEG // 10, D), lambda i: (i, 0)),
    out_shape=jax.ShapeDtypeStruct((NSEG, D), jnp.float32),
)


def kernel(v, segment_ids):
    seg = segment_ids.astype(jnp.int32)
    m, s = _sc_call(v, seg)
    return _finalize(m.reshape(NSEG, D), s.reshape(NSEG, D))


# speedup vs baseline: 1.2564x; 1.2397x over previous
import functools

import jax
import jax.numpy as jnp
import numpy as np
from jax import lax
from jax.experimental import pallas as pl
from jax.experimental.pallas import tpu as pltpu
from jax.experimental.pallas import tpu_sc as plsc

N = 320000
D = 128
NSEG = 10000
GAMMA = 0.01
INVG = float(np.float32(1.0) / np.float32(GAMMA))

NC = 2
NS = 16
NW = NC * NS
RPW = N // NW

C = 256
SCAP = 64
WIN = 2048
NVEC = 8

B64LN2 = 44.3614195558365
EPSU = 5.421010862427522e-32


def _sc_body(v_hbm, seg_hbm, u_hbm, rowb, idb, rowb2, idb2, idwin,
             id8, ustage, bounds, semr0, semi0, semr1, semi1):
    wid = lax.axis_index("c") * NS + lax.axis_index("s")

    zerov = jnp.zeros((16,), jnp.float32)

    def scan_window(p, b):
        p = pl.multiple_of(p, 8)
        pltpu.sync_copy(seg_hbm.at[pl.ds(p, WIN)], idwin)
        big = jnp.int32(10**9)

        def group(off, found):
            va = idwin[pl.ds(off, 16)]
            vb = idwin[pl.ds(off + 1, 16)]
            neq = va != vb
            io = lax.iota(jnp.int32, 16)
            cand = jnp.min(jnp.where(neq, io, big))
            pos = p + off + cand + 1
            ok = (cand < big) & (pos >= b)
            return jnp.where(ok, jnp.minimum(found, pos), found)

        def grp_k(k, found):
            return group(k * 16, found)

        found = lax.fori_loop(0, (WIN - 17) // 16 + 1, grp_k, jnp.int32(N))
        return group(jnp.int32(WIN - 17), found)

    def find_change(b):
        p0 = jnp.minimum(b - 8, N - WIN)
        f0 = scan_window(p0, b)

        def cond(st):
            return (st[1] == N) & (st[0] < N - WIN)

        def step(st):
            pn = jnp.minimum(st[0] + (WIN - 8), N - WIN)
            return (pn, scan_window(pn, b))

        return lax.while_loop(cond, step, (p0, f0))[1]

    b_lo = jnp.where(wid == 0, 8, wid * RPW)
    b_hi = jnp.where(wid == NW - 1, 8, (wid + 1) * RPW)
    a_start = jnp.where(wid == 0, 0, find_change(b_lo))
    a_end = jnp.where(wid == NW - 1, N, find_change(b_hi))

    def seg_at(r):
        r = jnp.minimum(r, N - 1)
        r8 = pl.multiple_of(jnp.minimum((r // 8) * 8, N - 16), 8)
        pltpu.sync_copy(seg_hbm.at[pl.ds(r8, 16)], id8.at[pl.ds(0, 16)])
        return id8[pl.ds(r - r8, 16)][0]

    own_lo = jnp.where(wid == 0, 0,
                       jnp.where(a_start >= N, NSEG, seg_at(a_start)))
    own_hi = jnp.where((wid == NW - 1) | (a_end >= N), NSEG, seg_at(a_end))

    def flush_full(base):
        o = pl.multiple_of(base * D, 8)
        pltpu.sync_copy(ustage, u_hbm.at[pl.ds(o, SCAP * D)])

    def maybe_flush(cnt, base):
        def f(st):
            flush_full(st[1])
            return (jnp.int32(0), st[1] + SCAP)

        return lax.cond(cnt == SCAP, f, lambda st: st, (cnt, base))

    def write_identity(cnt):
        for j in range(NVEC):
            ustage[pl.ds(cnt * D + j * 16, 16)] = zerov

    def fill_to(tid, cnt, base):
        def gcond(st):
            return st[0] + st[1] < tid

        def gbody(st):
            write_identity(st[0])
            return maybe_flush(st[0] + 1, st[1])

        return lax.while_loop(gcond, gbody, (cnt, base))

    def stage_put(sid, us, cnt, base):
        cnt, base = fill_to(sid, cnt, base)
        for j in range(NVEC):
            ustage[pl.ds(cnt * D + j * 16, 16)] = us[j]
        return maybe_flush(cnt + 1, base)

    A0 = (a_start // 8) * 8
    nchunks = (a_end - A0 + C - 1) // C

    def chunk_base(k):
        return pl.multiple_of(jnp.minimum(A0 + k * C, N - C), 8)

    def issue(k, rb, ib, semr, semi):
        r0c = chunk_base(k)
        pltpu.async_copy(v_hbm.at[pl.ds(r0c, C)], rb, semr)
        pltpu.async_copy(seg_hbm.at[pl.ds(r0c, C)], ib.at[pl.ds(0, C)], semi)

    def wait_dma(k, rb, ib, semr, semi):
        r0c = chunk_base(k)
        pltpu.make_async_copy(v_hbm.at[pl.ds(r0c, C)], rb, semr).wait()
        pltpu.make_async_copy(seg_hbm.at[pl.ds(r0c, C)], ib.at[pl.ds(0, C)],
                              semi).wait()

    def process_chunk(k, rb, ib, carry):
        r0 = A0 + k * C
        r0c = chunk_base(k)
        lo = jnp.maximum(a_start - r0c, r0 - r0c)
        hi = jnp.minimum(a_end - r0c, jnp.int32(C))

        def grp(g, nb):
            va = ib[pl.ds(g * 16, 16)]
            vb = ib[pl.ds(g * 16 + 1, 16)]
            msk = va != vb
            pos = lax.iota(jnp.int32, 16) + (g * 16 + 1)
            plsc.store_compressed(bounds.at[pl.ds(nb, 16)], pos, mask=msk)
            return nb + plsc.all_reduce_population_count(msk)[0]

        nb = lax.fori_loop(0, C // 16, grp, jnp.int32(0))

        def piece_body(q, cy):
            prev = cy[-1]
            b_q = jnp.where(q < nb, bounds[pl.ds(q, 16)][0], hi)
            s0 = jnp.maximum(prev, lo)
            e0 = jnp.minimum(b_q, hi)

            def do_piece(op):
                us = op[0:NVEC]
                cur, cnt, base = op[NVEC], op[NVEC + 1], op[NVEC + 2]
                pid = ib[pl.ds(s0, 16)][0]

                def fast1(i, acc):
                    return tuple(
                        acc[j] + jnp.exp(rb[i, pl.ds(j * 16, 16)] * INVG
                                         - B64LN2)
                        for j in range(NVEC))

                ul = lax.fori_loop(s0, e0, fast1, (zerov,) * NVEC)

                def merge(st):
                    return tuple(us[j] + ul[j] for j in range(NVEC)) + st

                def flush_rep(st):
                    cnt2, base2 = lax.cond(
                        cur >= 0,
                        lambda st2: stage_put(cur, us, st2[0], st2[1]),
                        lambda st2: st2,
                        st,
                    )
                    return tuple(ul) + (cnt2, base2)

                out = lax.cond(pid == cur, merge, flush_rep, (cnt, base))
                return out[:NVEC] + (pid,) + out[NVEC:]

            core = lax.cond(e0 > s0, do_piece, lambda op: op, cy[:-1])
            return core + (b_q,)

        cy = lax.fori_loop(0, nb + 1, piece_body, carry + (jnp.int32(0),))
        return cy[:-1]

    carry0 = (zerov,) * NVEC + (jnp.int32(-1), jnp.int32(0), own_lo)

    npairs = (nchunks + 1) // 2
    issue(jnp.int32(0), rowb, idb, semr0, semi0)

    def pair_body(g, carry):
        k0 = 2 * g
        issue(k0 + 1, rowb2, idb2, semr1, semi1)
        wait_dma(k0, rowb, idb, semr0, semi0)
        carry = process_chunk(k0, rowb, idb, carry)
        issue(k0 + 2, rowb, idb, semr0, semi0)
        wait_dma(k0 + 1, rowb2, idb2, semr1, semi1)
        carry = process_chunk(k0 + 1, rowb2, idb2, carry)
        return carry

    carry = lax.fori_loop(0, npairs, pair_body, carry0)
    wait_dma(2 * npairs, rowb, idb, semr0, semi0)

    us = carry[0:NVEC]
    cur, cnt, base = carry[NVEC], carry[NVEC + 1], carry[NVEC + 2]

    cnt, base = lax.cond(
        cur >= 0,
        lambda st: stage_put(cur, us, st[0], st[1]),
        lambda st: st,
        (cnt, base),
    )
    cnt, base = fill_to(own_hi, cnt, base)

    def blk_cond(off):
        return off + 8 <= cnt

    def blk_body(off):
        so = pl.multiple_of(off * D, 8)
        do = pl.multiple_of((base + off) * D, 8)
        pltpu.sync_copy(ustage.at[pl.ds(so, 8 * D)], u_hbm.at[pl.ds(do, 8 * D)])
        return off + 8

    off = lax.while_loop(blk_cond, blk_body, jnp.int32(0))

    def one_cond(off):
        return off < cnt

    def one_body(off):
        so = pl.multiple_of(off * D, 8)
        do = pl.multiple_of((base + off) * D, 8)
        pltpu.sync_copy(ustage.at[pl.ds(so, D)], u_hbm.at[pl.ds(do, D)])
        return off + 1

    lax.while_loop(one_cond, one_body, off)


_sc_call = functools.partial(
    pl.kernel,
    out_type=[
        jax.ShapeDtypeStruct((NSEG * D,), jnp.float32),
    ],
    mesh=plsc.VectorSubcoreMesh(
        core_axis_name="c", subcore_axis_name="s", num_cores=NC,
        num_subcores=NS),
    compiler_params=pltpu.CompilerParams(needs_layout_passes=False),
    scratch_types=[
        pltpu.VMEM((C, D), jnp.float32),
        pltpu.VMEM((C + 16,), jnp.int32),
        pltpu.VMEM((C, D), jnp.float32),
        pltpu.VMEM((C + 16,), jnp.int32),
        pltpu.VMEM((WIN,), jnp.int32),
        pltpu.VMEM((32,), jnp.int32),
        pltpu.VMEM((SCAP * D,), jnp.float32),
        pltpu.VMEM((C + 32,), jnp.int32),
        pltpu.SemaphoreType.DMA,
        pltpu.SemaphoreType.DMA,
        pltpu.SemaphoreType.DMA,
        pltpu.SemaphoreType.DMA,
    ],
)(_sc_body)


def _fin_body(u_ref, o_ref):
    u = u_ref[...]
    o_ref[...] = jnp.float32(GAMMA) * (
        jnp.float32(B64LN2) + jnp.log(u + jnp.float32(EPSU)))


_finalize = pl.pallas_call(
    _fin_body,
    grid=(10,),
    in_specs=[
        pl.BlockSpec((NSEG // 10, D), lambda i: (i, 0)),
    ],
    out_specs=pl.BlockSpec((NSEG // 10, D), lambda i: (i, 0)),
    out_shape=jax.ShapeDtypeStruct((NSEG, D), jnp.float32),
)


def kernel(v, segment_ids):
    seg = segment_ids.astype(jnp.int32)
    (u,) = _sc_call(v, seg)
    return _finalize(u.reshape(NSEG, D))
